# baseline (speedup 1.0000x reference)
import functools
import math

import jax
import jax.numpy as jnp
from jax import lax
from jax.experimental import pallas as pl
from jax.experimental.pallas import tpu as pltpu
from jax.experimental.pallas import tpu_sc as plsc

D_MODEL = 64
SCALE = math.sqrt(D_MODEL)

_INFO = plsc.get_sparse_core_info()
NUM_WORKERS = _INFO.num_cores * _INFO.num_subcores

CHUNK = 128


def _emb_kernel(n_chunks, x_hbm, table_hbm, out_hbm, idx_v, rows_v, sem):
    wid = lax.axis_index("s") * _INFO.num_cores + lax.axis_index("c")
    base = wid * (n_chunks * CHUNK)

    def body(i, _):
        off = base + i * CHUNK
        pltpu.sync_copy(x_hbm.at[pl.ds(off, CHUNK)], idx_v)
        pltpu.async_copy(table_hbm.at[idx_v], rows_v, sem).wait()

        def scale_body(j, _):
            for k in range(D_MODEL // 16):
                rows_v[j, pl.ds(k * 16, 16)] = rows_v[j, pl.ds(k * 16, 16)] * SCALE
            return ()

        lax.fori_loop(0, CHUNK, scale_body, (), unroll=2)
        pltpu.sync_copy(rows_v, out_hbm.at[pl.ds(off, CHUNK)])
        return ()

    lax.fori_loop(0, n_chunks, body, ())


def kernel(x, table):
    orig_shape = x.shape
    xf = x.reshape(-1)
    b_total = xf.shape[0]
    assert b_total % (NUM_WORKERS * CHUNK) == 0
    n_chunks = b_total // (NUM_WORKERS * CHUNK)

    mesh = plsc.VectorSubcoreMesh(core_axis_name="c", subcore_axis_name="s")
    run = pl.kernel(
        functools.partial(_emb_kernel, n_chunks),
        out_type=jax.ShapeDtypeStruct((b_total, D_MODEL), jnp.float32),
        mesh=mesh,
        scratch_types=[
            pltpu.VMEM((CHUNK,), jnp.int32),
            pltpu.VMEM((CHUNK, D_MODEL), jnp.float32),
            pltpu.SemaphoreType.DMA,
        ],
        compiler_params=pltpu.CompilerParams(use_tc_tiling_on_sc=False),
    )
    out = run(xf, table)
    return out.reshape(*orig_shape, D_MODEL)


# speedup vs baseline: 1.1932x; 1.1932x over previous
import functools
import math

import jax
import jax.numpy as jnp
from jax import lax
from jax.experimental import pallas as pl
from jax.experimental.pallas import tpu as pltpu
from jax.experimental.pallas import tpu_sc as plsc

D_MODEL = 64
SCALE = math.sqrt(D_MODEL)

_INFO = plsc.get_sparse_core_info()
NUM_WORKERS = _INFO.num_cores * _INFO.num_subcores

CHUNK = 512


def _emb_kernel(n_chunks, x_hbm, table_hbm, out_hbm, idx_v, rows_v, sem):
    wid = lax.axis_index("s") * _INFO.num_cores + lax.axis_index("c")
    base = wid * (n_chunks * CHUNK)

    def body(i, _):
        off = base + i * CHUNK
        pltpu.sync_copy(x_hbm.at[pl.ds(off, CHUNK)], idx_v)
        pltpu.async_copy(table_hbm.at[idx_v], rows_v, sem).wait()

        def scale_body(j, _):
            for k in range(D_MODEL // 16):
                rows_v[j, pl.ds(k * 16, 16)] = rows_v[j, pl.ds(k * 16, 16)] * SCALE
            return ()

        lax.fori_loop(0, CHUNK, scale_body, (), unroll=2)
        pltpu.sync_copy(rows_v, out_hbm.at[pl.ds(off, CHUNK)])
        return ()

    lax.fori_loop(0, n_chunks, body, ())


def kernel(x, table):
    orig_shape = x.shape
    xf = x.reshape(-1)
    b_total = xf.shape[0]
    assert b_total % (NUM_WORKERS * CHUNK) == 0
    n_chunks = b_total // (NUM_WORKERS * CHUNK)

    mesh = plsc.VectorSubcoreMesh(core_axis_name="c", subcore_axis_name="s")
    run = pl.kernel(
        functools.partial(_emb_kernel, n_chunks),
        out_type=jax.ShapeDtypeStruct((b_total, D_MODEL), jnp.float32),
        mesh=mesh,
        scratch_types=[
            pltpu.VMEM((CHUNK,), jnp.int32),
            pltpu.VMEM((CHUNK, D_MODEL), jnp.float32),
            pltpu.SemaphoreType.DMA,
        ],
        compiler_params=pltpu.CompilerParams(use_tc_tiling_on_sc=False),
    )
    out = run(xf, table)
    return out.reshape(*orig_shape, D_MODEL)


# speedup vs baseline: 1.3757x; 1.1529x over previous
import functools
import math

import jax
import jax.numpy as jnp
from jax import lax
from jax.experimental import pallas as pl
from jax.experimental.pallas import tpu as pltpu
from jax.experimental.pallas import tpu_sc as plsc

D_MODEL = 64
SCALE = math.sqrt(D_MODEL)

_INFO = plsc.get_sparse_core_info()
NUM_WORKERS = _INFO.num_cores * _INFO.num_subcores

CHUNK = 400
NBUF = 4


def _emb_kernel(n_chunks, x_hbm, table_hbm, out_hbm, idx_v, rows_v, *sems):
    gsem = sems[0:NBUF]
    isem = sems[NBUF:2 * NBUF]
    osem = sems[2 * NBUF:3 * NBUF]
    wid = lax.axis_index("s") * _INFO.num_cores + lax.axis_index("c")
    base = wid * (n_chunks * CHUNK)

    def idx_start(c, b):
        pltpu.async_copy(x_hbm.at[pl.ds(base + c * CHUNK, CHUNK)],
                         idx_v.at[b], isem[b])

    def idx_wait(b):
        pltpu.make_async_copy(x_hbm.at[pl.ds(base, CHUNK)],
                              idx_v.at[b], isem[b]).wait()

    def gather_start(b):
        pltpu.async_copy(table_hbm.at[idx_v.at[b]], rows_v.at[b], gsem[b])

    def gather_wait(b):
        pltpu.make_async_copy(table_hbm.at[idx_v.at[b]],
                              rows_v.at[b], gsem[b]).wait()

    def ostore_start(c, b):
        pltpu.async_copy(rows_v.at[b],
                         out_hbm.at[pl.ds(base + c * CHUNK, CHUNK)], osem[b])

    def ostore_wait(b):
        pltpu.make_async_copy(rows_v.at[b],
                              out_hbm.at[pl.ds(base, CHUNK)], osem[b]).wait()

    def scale(b):
        def sb(j, _):
            for k in range(D_MODEL // 16):
                rows_v[b, j, pl.ds(k * 16, 16)] = (
                    rows_v[b, j, pl.ds(k * 16, 16)] * SCALE)
            return ()

        lax.fori_loop(0, CHUNK, sb, (), unroll=8)

    def do_chunk(i, b, launch_gather, wait_ostore, launch_idx):
        bj = (b + 2) % NBUF
        if launch_gather:
            if wait_ostore:
                ostore_wait(bj)
            idx_wait(bj)
            gather_start(bj)
        gather_wait(b)
        if launch_idx:
            idx_start(i + 4, b)
        scale(b)
        ostore_start(i, b)

    pltpu.sync_copy(x_hbm.at[pl.ds(base, CHUNK)], idx_v.at[0])
    gather_start(0)
    pltpu.sync_copy(x_hbm.at[pl.ds(base + CHUNK, CHUNK)], idx_v.at[1])
    gather_start(1)
    idx_start(2, 2)
    idx_start(3, 3)

    do_chunk(0, 0, True, False, True)
    do_chunk(1, 1, True, False, True)
    do_chunk(2, 2, True, True, True)
    do_chunk(3, 3, True, True, True)

    steps = n_chunks // NBUF

    def body(s, _):
        i0 = s * NBUF
        for b in range(NBUF):
            do_chunk(i0 + b, b, True, True, True)
        return ()

    lax.fori_loop(1, steps - 1, body, ())

    n = n_chunks
    do_chunk(n - 4, 0, True, True, False)
    do_chunk(n - 3, 1, True, True, False)
    do_chunk(n - 2, 2, False, False, False)
    do_chunk(n - 1, 3, False, False, False)

    for b in range(NBUF):
        ostore_wait(b)


def kernel(x, table):
    orig_shape = x.shape
    xf = x.reshape(-1)
    b_total = xf.shape[0]
    assert b_total % (NUM_WORKERS * CHUNK) == 0
    n_chunks = b_total // (NUM_WORKERS * CHUNK)
    assert n_chunks % NBUF == 0 and n_chunks >= 2 * NBUF

    mesh = plsc.VectorSubcoreMesh(core_axis_name="c", subcore_axis_name="s")
    run = pl.kernel(
        functools.partial(_emb_kernel, n_chunks),
        out_type=jax.ShapeDtypeStruct((b_total, D_MODEL), jnp.float32),
        mesh=mesh,
        scratch_types=(
            [pltpu.VMEM((NBUF, CHUNK), jnp.int32),
             pltpu.VMEM((NBUF, CHUNK, D_MODEL), jnp.float32)]
            + [pltpu.SemaphoreType.DMA] * (3 * NBUF)
        ),
        compiler_params=pltpu.CompilerParams(use_tc_tiling_on_sc=False),
    )
    out = run(xf, table)
    return out.reshape(*orig_shape, D_MODEL)
